# baseline (speedup 1.0000x reference)
import jax
import jax.numpy as jnp
from jax import lax
from jax.experimental import pallas as pl
from jax.experimental.pallas import tpu as pltpu
from jax.experimental.pallas import tpu_sc as plsc
from jax.experimental import layout as jax_layout

BATCH = 16384
D = 16
NC = 2
NS = 16
NW = NC * NS
B_PER_W = BATCH // NW
CHUNK = 128
NCHUNK = B_PER_W // CHUNK
GROUPS = B_PER_W // 16


def _bpr_body(user_hbm, item_i_hbm, item_j_hbm, eu_hbm, ei_hbm, out_hbm,
              idx_u, idx_i, idx_j, u_rows, vi_rows, vj_rows, tot, sem):
    wid = lax.axis_index("s") * NC + lax.axis_index("c")
    base = wid * B_PER_W

    for k in range(NCHUNK):
        off = base + k * CHUNK
        pltpu.sync_copy(user_hbm.at[pl.ds(off, CHUNK)], idx_u.at[k])
        pltpu.sync_copy(item_i_hbm.at[pl.ds(off, CHUNK)], idx_i.at[k])
        pltpu.sync_copy(item_j_hbm.at[pl.ds(off, CHUNK)], idx_j.at[k])

    cps = []
    for k in range(NCHUNK):
        sl = pl.ds(k * CHUNK, CHUNK)
        cps.append(pltpu.async_copy(eu_hbm.at[idx_u.at[k]], u_rows.at[sl], sem))
        cps.append(pltpu.async_copy(ei_hbm.at[idx_i.at[k]], vi_rows.at[sl], sem))
        cps.append(pltpu.async_copy(ei_hbm.at[idx_j.at[k]], vj_rows.at[sl], sem))
    for cp in cps:
        cp.wait()

    lanes = lax.iota(jnp.int32, 16)

    def group_body(g, acc):
        rows = lanes + g * 16
        dot = jnp.zeros((16,), jnp.float32)
        for f in range(D):
            col = jnp.full((16,), f, jnp.int32)
            u_c = plsc.load_gather(u_rows, [rows, col])
            vi_c = plsc.load_gather(vi_rows, [rows, col])
            vj_c = plsc.load_gather(vj_rows, [rows, col])
            dot = dot + u_c * (vj_c - vi_c)
        sig = 1.0 / (1.0 + jnp.exp(-dot))
        return acc + sig

    total = lax.fori_loop(0, GROUPS, group_body, jnp.zeros((16,), jnp.float32))
    tot[...] = total
    pltpu.sync_copy(tot, out_hbm.at[wid])


@jax.jit
def _bpr(user, item_i, item_j, embed_user, embed_item):
    mesh = plsc.VectorSubcoreMesh(core_axis_name="c", subcore_axis_name="s")
    run = pl.kernel(
        _bpr_body,
        out_type=jax.ShapeDtypeStruct((NW, 16), jnp.float32),
        mesh=mesh,
        compiler_params=pltpu.CompilerParams(
            needs_layout_passes=False, use_tc_tiling_on_sc=False),
        scratch_types=[
            pltpu.VMEM((NCHUNK, CHUNK), jnp.int32),
            pltpu.VMEM((NCHUNK, CHUNK), jnp.int32),
            pltpu.VMEM((NCHUNK, CHUNK), jnp.int32),
            pltpu.VMEM((B_PER_W, D), jnp.float32),
            pltpu.VMEM((B_PER_W, D), jnp.float32),
            pltpu.VMEM((B_PER_W, D), jnp.float32),
            pltpu.VMEM((16,), jnp.float32),
            pltpu.SemaphoreType.DMA,
        ],
    )
    partials = run(user, item_i, item_j, embed_user, embed_item)
    return jnp.sum(partials)


_DCHUNK = 512
_NROWS = 1000000
_MAIN = (_NROWS // _DCHUNK) * _DCHUNK
_NMAIN = _MAIN // _DCHUNK
_REM = _NROWS - _MAIN
_UNROLL = 8


def _detile_body(src_hbm, tail_hbm, out_hbm, buf, rows):
    wid = lax.axis_index("s") * NC + lax.axis_index("c")
    lanes = lax.iota(jnp.int32, 16)

    def transpose_cols(width):
        def body(g, _):
            for j in range(_UNROLL):
                u = g * _UNROLL + j
                uv = jnp.full((16,), u, jnp.int32)
                col = plsc.load_gather(buf, [lanes, uv])
                plsc.store_scatter(rows, [uv, lanes], col)
            return 0
        lax.fori_loop(0, width // _UNROLL, body, 0)

    def do_chunk(base, width):
        pltpu.sync_copy(src_hbm.at[:, pl.ds(base, width)],
                        buf.at[:, pl.ds(0, width)])
        transpose_cols(width)
        pltpu.sync_copy(rows.at[pl.ds(0, width)],
                        out_hbm.at[pl.ds(base, width)])

    def main_body(k, _):
        kk = jnp.minimum(k, _NMAIN // NW - 1)
        do_chunk((kk * NW + wid) * _DCHUNK, _DCHUNK)
        return 0

    lax.fori_loop(0, _NMAIN // NW, main_body, 0)

    @pl.when(wid == 0)
    def _():
        do_chunk((_NMAIN - 1) * _DCHUNK, _DCHUNK)

    @pl.when(wid == 1)
    def _():
        pltpu.sync_copy(tail_hbm, buf.at[:, pl.ds(0, 128)])
        transpose_cols(64)
        pltpu.sync_copy(rows.at[pl.ds(0, 64)],
                        out_hbm.at[pl.ds(_MAIN, 64)])


def _detile(x_t, tail):
    mesh = plsc.VectorSubcoreMesh(core_axis_name="c", subcore_axis_name="s")
    run = pl.kernel(
        _detile_body,
        out_type=jax.ShapeDtypeStruct((_NROWS, D), jnp.float32),
        mesh=mesh,
        compiler_params=pltpu.CompilerParams(needs_layout_passes=False),
        scratch_types=[
            pltpu.VMEM((D, _DCHUNK), jnp.float32),
            pltpu.VMEM((_DCHUNK, D), jnp.float32),
        ],
    )
    return run(x_t, tail)


def _tail128(x_t):
    return jnp.pad(x_t[:, _MAIN:], ((0, 0), (0, 128 - _REM)))


def kernel(user, item_i, item_j, embed_user, embed_item):
    eu_t = embed_user.T
    ei_t = embed_item.T
    return _bpr(user, item_i, item_j,
                _detile(eu_t, _tail128(eu_t)), _detile(ei_t, _tail128(ei_t)))


# speedup vs baseline: 2.9525x; 2.9525x over previous
import jax
import jax.numpy as jnp
from jax import lax
from jax.experimental import pallas as pl
from jax.experimental.pallas import tpu as pltpu
from jax.experimental.pallas import tpu_sc as plsc
from jax.experimental import layout as jax_layout

BATCH = 16384
D = 16
NC = 2
NS = 16
NW = NC * NS
B_PER_W = BATCH // NW
CHUNK = 128
NCHUNK = B_PER_W // CHUNK
GROUPS = B_PER_W // 16


def _bpr_body(user_hbm, item_i_hbm, item_j_hbm, eu_hbm, ei_hbm, out_hbm,
              idx_u, idx_i, idx_j, u_rows, vi_rows, vj_rows, tot, sem):
    wid = lax.axis_index("s") * NC + lax.axis_index("c")
    base = wid * B_PER_W

    for k in range(NCHUNK):
        off = base + k * CHUNK
        pltpu.sync_copy(user_hbm.at[pl.ds(off, CHUNK)], idx_u.at[k])
        pltpu.sync_copy(item_i_hbm.at[pl.ds(off, CHUNK)], idx_i.at[k])
        pltpu.sync_copy(item_j_hbm.at[pl.ds(off, CHUNK)], idx_j.at[k])

    cps = []
    for k in range(NCHUNK):
        sl = pl.ds(k * CHUNK, CHUNK)
        cps.append(pltpu.async_copy(eu_hbm.at[idx_u.at[k]], u_rows.at[sl], sem))
        cps.append(pltpu.async_copy(ei_hbm.at[idx_i.at[k]], vi_rows.at[sl], sem))
        cps.append(pltpu.async_copy(ei_hbm.at[idx_j.at[k]], vj_rows.at[sl], sem))
    for cp in cps:
        cp.wait()

    lanes = lax.iota(jnp.int32, 16)

    def group_body(g, acc):
        rows = lanes + g * 16
        dot = jnp.zeros((16,), jnp.float32)
        for f in range(D):
            col = jnp.full((16,), f, jnp.int32)
            u_c = plsc.load_gather(u_rows, [rows, col])
            vi_c = plsc.load_gather(vi_rows, [rows, col])
            vj_c = plsc.load_gather(vj_rows, [rows, col])
            dot = dot + u_c * (vj_c - vi_c)
        sig = 1.0 / (1.0 + jnp.exp(-dot))
        return acc + sig

    total = lax.fori_loop(0, GROUPS, group_body, jnp.zeros((16,), jnp.float32))
    tot[...] = total
    pltpu.sync_copy(tot, out_hbm.at[wid])


@jax.jit
def _bpr(user, item_i, item_j, embed_user, embed_item):
    mesh = plsc.VectorSubcoreMesh(core_axis_name="c", subcore_axis_name="s")
    run = pl.kernel(
        _bpr_body,
        out_type=jax.ShapeDtypeStruct((NW, 16), jnp.float32),
        mesh=mesh,
        compiler_params=pltpu.CompilerParams(
            needs_layout_passes=False, use_tc_tiling_on_sc=False),
        scratch_types=[
            pltpu.VMEM((NCHUNK, CHUNK), jnp.int32),
            pltpu.VMEM((NCHUNK, CHUNK), jnp.int32),
            pltpu.VMEM((NCHUNK, CHUNK), jnp.int32),
            pltpu.VMEM((B_PER_W, D), jnp.float32),
            pltpu.VMEM((B_PER_W, D), jnp.float32),
            pltpu.VMEM((B_PER_W, D), jnp.float32),
            pltpu.VMEM((16,), jnp.float32),
            pltpu.SemaphoreType.DMA,
        ],
    )
    partials = run(user, item_i, item_j, embed_user, embed_item)
    return jnp.sum(partials)


def _row_major(x):
    lay = jax_layout.Layout(major_to_minor=(0, 1))
    return jax_layout.with_layout_constraint(x, lay)


def kernel(user, item_i, item_j, embed_user, embed_item):
    return _bpr(user, item_i, item_j,
                _row_major(embed_user), _row_major(embed_item))
